# baseline (speedup 1.0000x reference)
import functools

import jax
import jax.numpy as jnp
from jax import lax
from jax.experimental import pallas as pl
from jax.experimental.pallas import tpu as pltpu
from jax.experimental.pallas import tpu_sc as plsc

NC = 2
NS = 16
L = 16
NW = NC * NS

B = 16384
D = 32
BPW = B // NW
IDX_CH = 128
NCH = BPW // IDX_CH


def _body(uid_hbm, iid_hbm, ut_hbm, it_hbm, out_hbm,
          uidx_v, iidx_v, urows_v, irows_v, out_v, sem):
    c = lax.axis_index("c")
    s = lax.axis_index("s")
    wid = s * NC + c
    base = wid * BPW

    pltpu.sync_copy(uid_hbm.at[pl.ds(wid * NCH, NCH)], uidx_v)
    pltpu.sync_copy(iid_hbm.at[pl.ds(wid * NCH, NCH)], iidx_v)

    copies = []
    for j in range(NCH):
        copies.append(pltpu.async_copy(
            ut_hbm.at[uidx_v.at[j]],
            urows_v.at[pl.ds(j * IDX_CH, IDX_CH)], sem))
        copies.append(pltpu.async_copy(
            it_hbm.at[iidx_v.at[j]],
            irows_v.at[pl.ds(j * IDX_CH, IDX_CH)], sem))
    for cp in copies:
        cp.wait()

    lane = lax.broadcasted_iota(jnp.int32, (L,), 0)

    def chunk(ci, carry):
        row = ci * L + lane
        acc = jnp.zeros((L,), jnp.float32)
        for k in range(D):
            col = lax.bitwise_and(lane + k, D - 1)
            a = plsc.load_gather(urows_v, [row, col])
            b = plsc.load_gather(irows_v, [row, col])
            acc = acc + a * b
        out_v[pl.ds(ci * L, L)] = acc
        return carry

    lax.fori_loop(0, BPW // L, chunk, 0)

    pltpu.sync_copy(out_v, out_hbm.at[pl.ds(base, BPW)])


def kernel(user_ids, item_ids, user_table, item_table):
    uid = user_ids.astype(jnp.int32).reshape(NW * NCH, IDX_CH)
    iid = item_ids.astype(jnp.int32).reshape(NW * NCH, IDX_CH)
    mesh = plsc.VectorSubcoreMesh(core_axis_name="c", subcore_axis_name="s")
    run = functools.partial(
        pl.kernel,
        mesh=mesh,
        compiler_params=pltpu.CompilerParams(
            needs_layout_passes=False, use_tc_tiling_on_sc=False),
        out_type=jax.ShapeDtypeStruct((B,), jnp.float32),
        scratch_types=[
            pltpu.VMEM((NCH, IDX_CH), jnp.int32),
            pltpu.VMEM((NCH, IDX_CH), jnp.int32),
            pltpu.VMEM((BPW, D), jnp.float32),
            pltpu.VMEM((BPW, D), jnp.float32),
            pltpu.VMEM((BPW,), jnp.float32),
            pltpu.SemaphoreType.DMA,
        ],
    )(_body)
    return run(uid, iid, user_table, item_table)


# speedup vs baseline: 3.7135x; 3.7135x over previous
import functools

import jax
import jax.numpy as jnp
from jax import lax
from jax.experimental import pallas as pl
from jax.experimental.pallas import tpu as pltpu
from jax.experimental.pallas import tpu_sc as plsc

NC = 2
NS = 16
L = 16
NW = NC * NS

B = 16384
D = 32
BPW = B // NW
GN = 8
NG = BPW // GN


def _body(uid_hbm, iid_hbm, utT_hbm, itT_hbm, out_hbm,
          uids_v, iids_v, ubufs, ibufs, out_v, sem):
    c = lax.axis_index("c")
    s = lax.axis_index("s")
    wid = s * NC + c

    pltpu.sync_copy(uid_hbm.at[wid], uids_v)
    pltpu.sync_copy(iid_hbm.at[wid], iids_v)

    lane = lax.broadcasted_iota(jnp.int32, (L,), 0)
    vq = lane & 7
    dhalf = (lane >> 3) << 4

    def zero(ci, carry):
        out_v[pl.ds(ci * L, L)] = jnp.zeros((L,), jnp.float32)
        return carry

    lax.fori_loop(0, BPW // L, zero, 0)

    def group(g, carry):
        base = g * GN
        gu = plsc.load_gather(uids_v, [base + vq])
        gi = plsc.load_gather(iids_v, [base + vq])
        jus = (gu >> 7) * 128
        jis = (gi >> 7) * 128
        for q in range(GN):
            ju = jus[q]
            ji = jis[q]
            pltpu.async_copy(
                utT_hbm.at[:, pl.ds(pl.multiple_of(ju, 128), 128)],
                ubufs.at[q], sem)
            pltpu.async_copy(
                itT_hbm.at[:, pl.ds(pl.multiple_of(ji, 128), 128)],
                ibufs.at[q], sem)
        for q in range(GN):
            pltpu.make_async_copy(
                utT_hbm.at[:, pl.ds(0, 128)], ubufs.at[q], sem).wait()
            pltpu.make_async_copy(
                utT_hbm.at[:, pl.ds(0, 128)], ibufs.at[q], sem).wait()

        cu = gu & 127
        ci = gi & 127
        acc = jnp.zeros((L,), jnp.float32)
        for k in range(L):
            dk = dhalf + k
            au = plsc.load_gather(ubufs, [vq, dk, cu])
            ai = plsc.load_gather(ibufs, [vq, dk, ci])
            acc = acc + au * ai
        oidx = base + vq
        plsc.addupdate_scatter(out_v, [oidx], acc, mask=lane < 8)
        plsc.addupdate_scatter(out_v, [oidx], acc, mask=lane >= 8)
        return carry

    lax.fori_loop(0, NG, group, 0)

    pltpu.sync_copy(out_v, out_hbm.at[pl.ds(wid * BPW, BPW)])


def kernel(user_ids, item_ids, user_table, item_table):
    uid = user_ids.astype(jnp.int32).reshape(NW, BPW)
    iid = item_ids.astype(jnp.int32).reshape(NW, BPW)
    utT = user_table.T
    itT = item_table.T
    mesh = plsc.VectorSubcoreMesh(core_axis_name="c", subcore_axis_name="s")
    run = functools.partial(
        pl.kernel,
        mesh=mesh,
        compiler_params=pltpu.CompilerParams(needs_layout_passes=False),
        out_type=jax.ShapeDtypeStruct((B,), jnp.float32),
        scratch_types=[
            pltpu.VMEM((BPW,), jnp.int32),
            pltpu.VMEM((BPW,), jnp.int32),
            pltpu.VMEM((GN, D, 128), jnp.float32),
            pltpu.VMEM((GN, D, 128), jnp.float32),
            pltpu.VMEM((BPW,), jnp.float32),
            pltpu.SemaphoreType.DMA,
        ],
    )(_body)
    return run(uid, iid, utT, itT)


# speedup vs baseline: 4.5364x; 1.2216x over previous
import functools

import jax
import jax.numpy as jnp
from jax import lax
from jax.experimental import pallas as pl
from jax.experimental.pallas import tpu as pltpu
from jax.experimental.pallas import tpu_sc as plsc

NC = 2
NS = 16
L = 16
NW = NC * NS

B = 16384
D = 32
BPW = B // NW
GN = 8
NG = BPW // GN


def _body(uid_hbm, iid_hbm, utT_hbm, itT_hbm, out_hbm,
          uids_v, iids_v, ubufs, ibufs, out_v, sem):
    c = lax.axis_index("c")
    s = lax.axis_index("s")
    wid = s * NC + c

    pltpu.sync_copy(uid_hbm.at[wid], uids_v)
    pltpu.sync_copy(iid_hbm.at[wid], iids_v)

    lane = lax.broadcasted_iota(jnp.int32, (L,), 0)
    vq4 = lane & 3
    dgrp = lane >> 2

    def zero(ci, carry):
        out_v[pl.ds(ci * L, L)] = jnp.zeros((L,), jnp.float32)
        return carry

    lax.fori_loop(0, BPW // L, zero, 0)

    HG = GN // 2
    NH = BPW // HG

    def fire_half(h):
        sb = lax.rem(h, 2) * HG
        gu = plsc.load_gather(uids_v, [h * HG + vq4])
        gi = plsc.load_gather(iids_v, [h * HG + vq4])
        jus = (gu >> 7) * 128
        jis = (gi >> 7) * 128
        for q in range(HG):
            pltpu.async_copy(
                utT_hbm.at[:, pl.ds(pl.multiple_of(jus[q], 128), 128)],
                ubufs.at[sb + q], sem)
            pltpu.async_copy(
                itT_hbm.at[:, pl.ds(pl.multiple_of(jis[q], 128), 128)],
                ibufs.at[sb + q], sem)

    def compute_half(h):
        base = h * HG
        sb = lax.rem(h, 2) * HG
        for q in range(2 * HG):
            pltpu.make_async_copy(
                utT_hbm.at[:, pl.ds(0, 128)], ubufs.at[0], sem).wait()
        gu = plsc.load_gather(uids_v, [base + vq4])
        gi = plsc.load_gather(iids_v, [base + vq4])
        cu = gu & 127
        ci = gi & 127
        slot = sb + vq4
        acc = jnp.zeros((L,), jnp.float32)
        for k in range(D // 4):
            dk = dgrp * (D // 4) + k
            au = plsc.load_gather(ubufs, [slot, dk, cu])
            ai = plsc.load_gather(ibufs, [slot, dk, ci])
            acc = acc + au * ai
        oidx = base + vq4
        for m in range(4):
            plsc.addupdate_scatter(out_v, [oidx], acc, mask=dgrp == m)

    fire_half(0)

    def half(h, carry):
        fire_half(h + 1)
        compute_half(h)
        return carry

    lax.fori_loop(0, NH - 1, half, 0)
    compute_half(NH - 1)

    pltpu.sync_copy(out_v, out_hbm.at[pl.ds(wid * BPW, BPW)])


def kernel(user_ids, item_ids, user_table, item_table):
    uid = user_ids.astype(jnp.int32).reshape(NW, BPW)
    iid = item_ids.astype(jnp.int32).reshape(NW, BPW)
    utT = user_table.T
    itT = item_table.T
    mesh = plsc.VectorSubcoreMesh(core_axis_name="c", subcore_axis_name="s")
    run = functools.partial(
        pl.kernel,
        mesh=mesh,
        compiler_params=pltpu.CompilerParams(needs_layout_passes=False),
        out_type=jax.ShapeDtypeStruct((B,), jnp.float32),
        scratch_types=[
            pltpu.VMEM((BPW,), jnp.int32),
            pltpu.VMEM((BPW,), jnp.int32),
            pltpu.VMEM((GN, D, 128), jnp.float32),
            pltpu.VMEM((GN, D, 128), jnp.float32),
            pltpu.VMEM((BPW,), jnp.float32),
            pltpu.SemaphoreType.DMA,
        ],
    )(_body)
    return run(uid, iid, utT, itT)
